# baseline (speedup 1.0000x reference)
import jax
import jax.numpy as jnp
from jax.experimental import pallas as pl
from jax.experimental.pallas import tpu as pltpu


def _scatter_body(gr, er, ir, emb_ref, id_ref, qe_ref, qi_ref, oe_ref, oi_ref):
    oe_ref[...] = emb_ref[...]
    oi_ref[...] = id_ref[...]


def kernel(embeddings, queue_emb, item_ids, queue_ids):
    C, D = queue_emb.shape
    B = item_ids.shape[0]

    order = jnp.argsort(item_ids).astype(jnp.int32)
    sids = jnp.take(item_ids, order)
    uniq = jnp.concatenate(
        [jnp.ones((1,), dtype=bool), sids[1:] != sids[:-1]], axis=0)

    q_order = jnp.argsort(queue_ids)
    q_sorted = jnp.take(queue_ids, q_order)
    pos = jnp.clip(jnp.searchsorted(q_sorted, sids), 0, C - 1)
    found = q_sorted[pos] == sids
    idx = jnp.where(found, jnp.take(q_order, pos), -1)

    valid_update = uniq & (idx >= 0)
    upd_row = jnp.where(valid_update, idx, C).astype(jnp.int32)

    new_mask = uniq & (idx < 0)
    enq_pos = jnp.mod(jnp.cumsum(new_mask.astype(jnp.int32)) - 1, C)
    enq_row = jnp.where(new_mask, enq_pos, C).astype(jnp.int32)

    gather_rows = jnp.concatenate([order, order])
    emb_rows = jnp.concatenate([upd_row, enq_row])
    ids_rows = jnp.concatenate([jnp.full((B,), C, jnp.int32), enq_row])

    qemb_pad = jnp.concatenate(
        [queue_emb, jnp.zeros((1, D), queue_emb.dtype)], axis=0).reshape(C + 1, 1, D)
    qids_pad = jnp.concatenate(
        [queue_ids, jnp.full((1,), -1, queue_ids.dtype)], axis=0).reshape(C + 1, 1, 1)
    emb3 = embeddings.reshape(B, 1, D)
    ids3 = item_ids.reshape(B, 1, 1)

    grid_spec = pltpu.PrefetchScalarGridSpec(
        num_scalar_prefetch=3,
        grid=(2 * B,),
        in_specs=[
            pl.BlockSpec((1, 1, D), lambda i, gr, er, ir: (gr[i], 0, 0)),
            pl.BlockSpec((1, 1, 1), lambda i, gr, er, ir: (gr[i], 0, 0)),
            pl.BlockSpec(memory_space=pl.ANY),
            pl.BlockSpec(memory_space=pl.ANY),
        ],
        out_specs=[
            pl.BlockSpec((1, 1, D), lambda i, gr, er, ir: (er[i], 0, 0)),
            pl.BlockSpec((1, 1, 1), lambda i, gr, er, ir: (ir[i], 0, 0)),
        ],
    )

    out_e, out_i = pl.pallas_call(
        _scatter_body,
        grid_spec=grid_spec,
        out_shape=[
            jax.ShapeDtypeStruct((C + 1, 1, D), queue_emb.dtype),
            jax.ShapeDtypeStruct((C + 1, 1, 1), queue_ids.dtype),
        ],
        input_output_aliases={5: 0, 6: 1},
    )(gather_rows, emb_rows, ids_rows, emb3, ids3, qemb_pad, qids_pad)

    return out_e[:C, 0], out_i[:C, 0, 0]
